# Initial kernel scaffold; baseline (speedup 1.0000x reference)
import jax
import jax.numpy as jnp
from jax.experimental import pallas as pl


def kernel(indices, pose_adjustment):
    raise NotImplementedError("write your pallas kernel here")


# speedup vs baseline: 6.3768x; 6.3768x over previous
import functools

import jax
import jax.numpy as jnp
from jax import lax
from jax.experimental import pallas as pl
from jax.experimental.pallas import tpu as pltpu
from jax.experimental.pallas import tpu_sc as plsc


def _se3_exp_table_kernel(x_ref, o_ref):
    def knot(k):
        b = 6 * k
        rx = x_ref[b + 0:b + 1, :]
        ry = x_ref[b + 1:b + 2, :]
        rz = x_ref[b + 2:b + 3, :]
        px = x_ref[b + 3:b + 4, :]
        py = x_ref[b + 4:b + 5, :]
        pz = x_ref[b + 5:b + 6, :]
        t2 = px * px + py * py + pz * pz
        t2s = jnp.maximum(t2, 1e-24)
        theta = jnp.sqrt(t2s)
        small = t2 < 1e-12
        half = 0.5 * theta
        sinc_half = jnp.where(small, 0.5 - t2 / 48.0, jnp.sin(half) / theta)
        qx = sinc_half * px
        qy = sinc_half * py
        qz = sinc_half * pz
        qw = jnp.cos(half)
        A = jnp.where(small, 0.5 - t2 / 24.0, (1.0 - jnp.cos(theta)) / t2s)
        B = jnp.where(small, 1.0 / 6.0 - t2 / 120.0,
                      (theta - jnp.sin(theta)) / (t2s * theta))
        c1 = 1.0 - B * t2
        dot = px * rx + py * ry + pz * rz
        cx = py * rz - pz * ry
        cy = pz * rx - px * rz
        cz = px * ry - py * rx
        tx = c1 * rx + A * cx + B * dot * px
        ty = c1 * ry + A * cy + B * dot * py
        tz = c1 * rz + A * cz + B * dot * pz
        return (tx, ty, tz, qx, qy, qz, qw)

    rows0 = knot(0)
    rows1 = knot(1)
    for i, v in enumerate(rows0):
        o_ref[i:i + 1, :] = v
    for i, v in enumerate(rows1):
        o_ref[7 + i:7 + i + 1, :] = v
    o_ref[14:16, :] = jnp.zeros((2, o_ref.shape[1]), jnp.float32)


def _make_gather(V, B):
    info = plsc.get_sparse_core_info()
    NC, NS = info.num_cores, info.num_subcores
    NW = NC * NS
    assert B % NW == 0
    b_per_w = B // NW
    CH = 128
    n_ch = b_per_w // CH
    assert n_ch * CH == b_per_w
    D = 16
    mesh = plsc.VectorSubcoreMesh(core_axis_name="c", subcore_axis_name="s")

    @functools.partial(
        pl.kernel,
        mesh=mesh,
        compiler_params=pltpu.CompilerParams(use_tc_tiling_on_sc=False),
        out_type=jax.ShapeDtypeStruct((B, D), jnp.float32),
        scratch_types=[
            pltpu.VMEM((n_ch, CH), jnp.int32),
            pltpu.VMEM((b_per_w, D), jnp.float32),
            pltpu.SemaphoreType.DMA,
        ],
    )
    def gather_k(table_hbm, idx_hbm, out_hbm, idx_v, rows_v, sem):
        wid = lax.axis_index("s") * NC + lax.axis_index("c")
        base = wid * b_per_w
        for j in range(n_ch):
            pltpu.sync_copy(idx_hbm.at[pl.ds(base + j * CH, CH)], idx_v.at[j])
        copies = []
        for j in range(n_ch):
            copies.append(
                pltpu.async_copy(
                    table_hbm.at[idx_v.at[j]],
                    rows_v.at[pl.ds(j * CH, CH)],
                    sem,
                ))
        for c in copies:
            c.wait()
        pltpu.sync_copy(rows_v, out_hbm.at[pl.ds(base, b_per_w)])

    return gather_k


def kernel(indices, pose_adjustment):
    V, K, _ = pose_adjustment.shape
    B = indices.shape[0]
    xi = pose_adjustment.transpose(1, 2, 0).reshape(K * 6, V)
    table_t = pl.pallas_call(
        _se3_exp_table_kernel,
        out_shape=jax.ShapeDtypeStruct((16, V), jnp.float32),
    )(xi)
    table = table_t.T
    idx2 = indices.reshape(-1, indices.shape[-1]) if indices.ndim > 1 else indices
    gathered = _make_gather(V, B)(table, idx2)
    return gathered[:, :14].reshape(B, K, 7)
